# baseline (speedup 1.0000x reference)
import functools

import jax
import jax.numpy as jnp
from jax.experimental import pallas as pl


def _tr_body(j_ref, w_ref):
    w_ref[...] = jnp.transpose(j_ref[...], (1, 0, 2)).astype(jnp.bfloat16)


def _matmul_body(x_ref, w_ref, h_ref, o_ref, *, Q, TN):
    t = pl.program_id(0)
    col0 = t * TN
    row_j = jax.lax.broadcasted_iota(jnp.int32, (w_ref.shape[0], 1), 0) // Q
    col_i = (jax.lax.broadcasted_iota(jnp.int32, (1, TN), 1) + col0) // Q
    mask = row_j < col_i
    wm = jnp.where(mask, w_ref[...], jnp.zeros((), w_ref.dtype))
    acc = jnp.dot(x_ref[...], wm, preferred_element_type=jnp.float32)
    o_ref[...] = acc + h_ref[...]


def kernel(X_oh, h_pos, J):
    M, L, Q = X_oh.shape
    LQ = L * Q

    BI, BK = 64, 64
    J3 = J.reshape(L, LQ, Q)
    W3 = pl.pallas_call(
        _tr_body,
        grid=(LQ // BK, L // BI),
        in_specs=[pl.BlockSpec((BI, BK, Q), lambda k, i: (i, k, 0))],
        out_specs=pl.BlockSpec((BK, BI, Q), lambda k, i: (k, i, 0)),
        out_shape=jax.ShapeDtypeStruct((LQ, L, Q), jnp.bfloat16),
    )(J3)
    W = W3.reshape(LQ, LQ)

    Xf = X_oh.reshape(M, LQ).astype(jnp.bfloat16)
    hf = h_pos.reshape(1, LQ)

    TN = 128
    n_col = LQ // TN

    out = pl.pallas_call(
        functools.partial(_matmul_body, Q=Q, TN=TN),
        grid=(n_col,),
        in_specs=[
            pl.BlockSpec((M, LQ), lambda t: (0, 0)),
            pl.BlockSpec((LQ, TN), lambda t: (0, t)),
            pl.BlockSpec((1, TN), lambda t: (0, t)),
        ],
        out_specs=pl.BlockSpec((M, TN), lambda t: (0, t)),
        out_shape=jax.ShapeDtypeStruct((M, LQ), jnp.float32),
    )(Xf, W, hf)
    return out.reshape(M, L, Q)


# speedup vs baseline: 1.7140x; 1.7140x over previous
import functools

import jax
import jax.numpy as jnp
from jax.experimental import pallas as pl
from jax.experimental.pallas import tpu as pltpu


def _body(x_ref, j_ref, h_ref, o_ref, xbf, wt, *, Q, BI, LQ):
    t = pl.program_id(0)

    @pl.when(t == 0)
    def _():
        xbf[...] = x_ref[...].astype(jnp.bfloat16)

    for il in range(BI):
        wt[:, il * Q:(il + 1) * Q] = j_ref[il].astype(jnp.bfloat16)

    row_j = jax.lax.broadcasted_iota(jnp.int32, (LQ, 1), 0) // Q
    col_i = t * BI + jax.lax.broadcasted_iota(jnp.int32, (1, BI * Q), 1) // Q
    wm = jnp.where(row_j < col_i, wt[...], jnp.zeros((), jnp.bfloat16))

    acc = jnp.dot(xbf[...], wm, preferred_element_type=jnp.float32)
    o_ref[0] = acc + h_ref[0]


def kernel(X_oh, h_pos, J):
    M, L, Q = X_oh.shape
    LQ = L * Q
    BI = 8
    TN = BI * Q
    n_col = L // BI
    J3 = J.reshape(L, LQ, Q)
    Xf = X_oh.reshape(M, LQ)
    hf = h_pos.reshape(n_col, 1, TN)

    out = pl.pallas_call(
        functools.partial(_body, Q=Q, BI=BI, LQ=LQ),
        grid=(n_col,),
        in_specs=[
            pl.BlockSpec((M, LQ), lambda t: (0, 0)),
            pl.BlockSpec((BI, LQ, Q), lambda t: (t, 0, 0)),
            pl.BlockSpec((1, 1, TN), lambda t: (t, 0, 0)),
        ],
        out_specs=pl.BlockSpec((1, M, TN), lambda t: (t, 0, 0)),
        out_shape=jax.ShapeDtypeStruct((n_col, M, TN), jnp.float32),
        scratch_shapes=[
            pltpu.VMEM((M, LQ), jnp.bfloat16),
            pltpu.VMEM((LQ, TN), jnp.bfloat16),
        ],
    )(Xf, J3, hf)
    return out.transpose(1, 0, 2).reshape(M, L, Q)


# speedup vs baseline: 3.0516x; 1.7804x over previous
import functools

import jax
import jax.numpy as jnp
from jax.experimental import pallas as pl
from jax.experimental.pallas import tpu as pltpu


def _body(x_ref, j_ref, h_ref, o_ref, xbf, wt, *, Q, L, BI, LQ):
    t = pl.program_id(0)

    @pl.when(t == 0)
    def _():
        xbf[...] = x_ref[...].astype(jnp.bfloat16)

    for il in range(BI):
        jb = j_ref[il]
        slab = jnp.concatenate(
            [jb[:, u * Q:(u + 1) * Q] for u in range(Q)], axis=0
        )
        wt[:, il * Q:(il + 1) * Q] = slab.astype(jnp.bfloat16)

    row_j = jax.lax.broadcasted_iota(jnp.int32, (LQ, 1), 0) % L
    col_i = t * BI + jax.lax.broadcasted_iota(jnp.int32, (1, BI * Q), 1) // Q
    wm = jnp.where(row_j < col_i, wt[...], jnp.zeros((), jnp.bfloat16))

    acc = jnp.dot(xbf[...], wm, preferred_element_type=jnp.float32)
    o_ref[0] = acc + h_ref[0]


def kernel(X_oh, h_pos, J):
    M, L, Q = X_oh.shape
    LQ = L * Q
    BI = 8
    TN = BI * Q
    n_col = L // BI

    J4 = J.reshape(L, L, Q * Q)
    Xp = X_oh.transpose(0, 2, 1).reshape(M, LQ)
    hf = h_pos.reshape(n_col, 1, TN)

    out = pl.pallas_call(
        functools.partial(_body, Q=Q, L=L, BI=BI, LQ=LQ),
        grid=(n_col,),
        in_specs=[
            pl.BlockSpec((M, LQ), lambda t: (0, 0)),
            pl.BlockSpec((BI, L, Q * Q), lambda t: (t, 0, 0)),
            pl.BlockSpec((1, 1, TN), lambda t: (t, 0, 0)),
        ],
        out_specs=pl.BlockSpec((1, M, TN), lambda t: (t, 0, 0)),
        out_shape=jax.ShapeDtypeStruct((n_col, M, TN), jnp.float32),
        scratch_shapes=[
            pltpu.VMEM((M, LQ), jnp.bfloat16),
            pltpu.VMEM((LQ, TN), jnp.bfloat16),
        ],
    )(Xp, J4, hf)
    return out.transpose(1, 0, 2).reshape(M, L, Q)


# speedup vs baseline: 3.0723x; 1.0068x over previous
import functools

import jax
import jax.numpy as jnp
from jax.experimental import pallas as pl
from jax.experimental.pallas import tpu as pltpu


def _body(x_ref, j_ref, h_ref, o_ref, wt, *, Q, L, BI, LQ):
    t = pl.program_id(0)

    row_j = jax.lax.broadcasted_iota(jnp.int32, (LQ, 1), 0) % L

    for il in range(BI):
        jb = j_ref[il]
        slab = jnp.concatenate(
            [jb[:, u * Q:(u + 1) * Q] for u in range(Q)], axis=0
        )
        keep = row_j < (t * BI + il)
        wt[:, il * Q:(il + 1) * Q] = jnp.where(keep, slab, 0.0).astype(
            jnp.bfloat16)

    acc = jnp.dot(x_ref[...], wt[...], preferred_element_type=jnp.float32)
    o_ref[0] = acc + h_ref[0]


def kernel(X_oh, h_pos, J):
    M, L, Q = X_oh.shape
    LQ = L * Q
    BI = 8
    TN = BI * Q
    n_col = L // BI

    J4 = J.reshape(L, L, Q * Q)
    Xp = X_oh.astype(jnp.bfloat16).transpose(0, 2, 1).reshape(M, LQ)
    hf = h_pos.reshape(n_col, 1, TN)

    out = pl.pallas_call(
        functools.partial(_body, Q=Q, L=L, BI=BI, LQ=LQ),
        grid=(n_col,),
        in_specs=[
            pl.BlockSpec((M, LQ), lambda t: (0, 0)),
            pl.BlockSpec((BI, L, Q * Q), lambda t: (t, 0, 0)),
            pl.BlockSpec((1, 1, TN), lambda t: (t, 0, 0)),
        ],
        out_specs=pl.BlockSpec((1, M, TN), lambda t: (t, 0, 0)),
        out_shape=jax.ShapeDtypeStruct((n_col, M, TN), jnp.float32),
        scratch_shapes=[
            pltpu.VMEM((LQ, TN), jnp.bfloat16),
        ],
    )(Xp, J4, hf)
    return out.transpose(1, 0, 2).reshape(M, L, Q)
